# baseline (speedup 1.0000x reference)
import jax
import jax.numpy as jnp
from jax.experimental import pallas as pl
from jax.experimental.pallas import tpu as pltpu

_B, _N, _D = 2, 4096, 4096


def _slice_body(idx_ref, x_ref, o_ref, sem0, sem1):
    i = idx_ref[0]
    c0 = pltpu.make_async_copy(x_ref.at[0, i, :], o_ref.at[0, :], sem0)
    c1 = pltpu.make_async_copy(x_ref.at[1, i, :], o_ref.at[1, :], sem1)
    c0.start()
    c1.start()
    c0.wait()
    c1.wait()


def kernel(x, indices):
    idx = jnp.asarray(indices, dtype=jnp.int32).reshape(1)
    return pl.pallas_call(
        _slice_body,
        in_specs=[
            pl.BlockSpec(memory_space=pltpu.MemorySpace.SMEM),
            pl.BlockSpec(memory_space=pltpu.MemorySpace.HBM),
        ],
        out_specs=pl.BlockSpec(memory_space=pltpu.MemorySpace.VMEM),
        out_shape=jax.ShapeDtypeStruct((_B, _D), jnp.float32),
        scratch_shapes=[pltpu.SemaphoreType.DMA, pltpu.SemaphoreType.DMA],
    )(idx, x)


# speedup vs baseline: 1.0257x; 1.0257x over previous
import jax
import jax.numpy as jnp
from jax.experimental import pallas as pl
from jax.experimental.pallas import tpu as pltpu

_B, _N, _D = 2, 4096, 4096


def _slice_body(idx_ref, x_ref, o_ref, sem):
    i = idx_ref[0]
    c = pltpu.make_async_copy(x_ref.at[:, i, :], o_ref, sem)
    c.start()
    c.wait()


def kernel(x, indices):
    idx = jnp.asarray(indices, dtype=jnp.int32).reshape(1)
    return pl.pallas_call(
        _slice_body,
        in_specs=[
            pl.BlockSpec(memory_space=pltpu.MemorySpace.SMEM),
            pl.BlockSpec(memory_space=pltpu.MemorySpace.HBM),
        ],
        out_specs=pl.BlockSpec(memory_space=pltpu.MemorySpace.VMEM),
        out_shape=jax.ShapeDtypeStruct((_B, _D), jnp.float32),
        scratch_shapes=[pltpu.SemaphoreType.DMA],
    )(idx, x)
